# Initial kernel scaffold; baseline (speedup 1.0000x reference)
import jax
import jax.numpy as jnp
from jax.experimental import pallas as pl


def kernel(x, weight):
    raise NotImplementedError("write your pallas kernel here")


# speedup vs baseline: 1.7963x; 1.7963x over previous
import functools

import jax
import jax.numpy as jnp
from jax import lax
from jax.experimental import pallas as pl
from jax.experimental.pallas import tpu as pltpu
from jax.experimental.pallas import tpu_sc as plsc

NC, NS = 2, 16
NW = NC * NS
D = 64
CHUNK = 512


@functools.partial(jax.jit, static_argnames=("total",))
def _gather(idx_flat, weight, total):
    b_per_w = total // NW
    n_chunks = b_per_w // CHUNK
    mesh = plsc.VectorSubcoreMesh(
        core_axis_name="c", subcore_axis_name="s", num_cores=NC, num_subcores=NS
    )

    @functools.partial(
        pl.kernel,
        mesh=mesh,
        out_type=jax.ShapeDtypeStruct((total, D), jnp.float32),
        scratch_types=[
            pltpu.VMEM((CHUNK,), jnp.int32),
            pltpu.VMEM((CHUNK, D), jnp.float32),
            pltpu.SemaphoreType.DMA,
        ],
        compiler_params=pltpu.CompilerParams(use_tc_tiling_on_sc=False),
    )
    def kern(idx_hbm, table_hbm, out_hbm, idx_v, rows_v, sem):
        wid = lax.axis_index("s") * NC + lax.axis_index("c")
        base = wid * b_per_w

        @pl.loop(0, n_chunks)
        def _(c):
            off = base + c * CHUNK
            pltpu.sync_copy(idx_hbm.at[pl.ds(off, CHUNK)], idx_v)
            pltpu.async_copy(table_hbm.at[idx_v], rows_v, sem).wait()
            pltpu.sync_copy(rows_v, out_hbm.at[pl.ds(off, CHUNK)])

    return kern(idx_flat, weight)


def kernel(x, weight):
    total = x.shape[0] * x.shape[1]
    idx_flat = x.reshape(total).astype(jnp.int32)
    out = _gather(idx_flat, weight, total)
    return out.reshape(x.shape[0], x.shape[1], D)


# speedup vs baseline: 1.8682x; 1.0400x over previous
import functools

import jax
import jax.numpy as jnp
from jax import lax
from jax.experimental import pallas as pl
from jax.experimental.pallas import tpu as pltpu
from jax.experimental.pallas import tpu_sc as plsc

NC, NS = 2, 16
NW = NC * NS
D = 64
CHUNK = 320
NBUF = 4


@functools.partial(jax.jit, static_argnames=("total",))
def _gather(idx_flat, weight, total):
    b_per_w = total // NW
    n_chunks = b_per_w // CHUNK
    n_waves = n_chunks // NBUF
    mesh = plsc.VectorSubcoreMesh(
        core_axis_name="c", subcore_axis_name="s", num_cores=NC, num_subcores=NS
    )

    @functools.partial(
        pl.kernel,
        mesh=mesh,
        out_type=jax.ShapeDtypeStruct((total, D), jnp.float32),
        scratch_types=[
            pltpu.VMEM((b_per_w,), jnp.int32),
            pltpu.VMEM((NBUF, CHUNK, D), jnp.float32),
            pltpu.SemaphoreType.DMA,
            pltpu.SemaphoreType.DMA,
            pltpu.SemaphoreType.DMA,
            pltpu.SemaphoreType.DMA,
            pltpu.SemaphoreType.DMA,
            pltpu.SemaphoreType.DMA,
            pltpu.SemaphoreType.DMA,
            pltpu.SemaphoreType.DMA,
        ],
        compiler_params=pltpu.CompilerParams(use_tc_tiling_on_sc=False),
    )
    def kern(idx_hbm, table_hbm, out_hbm, idx_v, rows_v, g0, g1, g2, g3, o0, o1, o2, o3):
        gsem = (g0, g1, g2, g3)
        osem = (o0, o1, o2, o3)
        wid = lax.axis_index("s") * NC + lax.axis_index("c")
        base = wid * b_per_w
        pltpu.sync_copy(idx_hbm.at[pl.ds(base, b_per_w)], idx_v)

        def gather_desc(c, b):
            return pltpu.make_async_copy(
                table_hbm.at[idx_v.at[pl.ds(c * CHUNK, CHUNK)]],
                rows_v.at[b],
                gsem[b],
            )

        def out_desc(c, b):
            return pltpu.make_async_copy(
                rows_v.at[b],
                out_hbm.at[pl.ds(base + c * CHUNK, CHUNK)],
                osem[b],
            )

        for b in range(NBUF):
            gather_desc(b, b).start()

        @pl.loop(0, n_waves - 1)
        def _(p):
            c = p * NBUF
            for b in range(NBUF):
                gather_desc(c + b, b).wait()
                out_desc(c + b, b).start()
            for b in range(NBUF):
                out_desc(c + b, b).wait()
                gather_desc(c + NBUF + b, b).start()

        c_last = (n_waves - 1) * NBUF
        for b in range(NBUF):
            gather_desc(c_last + b, b).wait()
            out_desc(c_last + b, b).start()
        for b in range(NBUF):
            out_desc(c_last + b, b).wait()

    return kern(idx_flat, weight)


def kernel(x, weight):
    total = x.shape[0] * x.shape[1]
    idx_flat = x.reshape(total).astype(jnp.int32)
    out = _gather(idx_flat, weight, total)
    return out.reshape(x.shape[0], x.shape[1], D)
